# Initial kernel scaffold; baseline (speedup 1.0000x reference)
import jax
import jax.numpy as jnp
from jax.experimental import pallas as pl


def kernel(edge_index, adj_values, uEmbeds):
    raise NotImplementedError("write your pallas kernel here")


# speedup vs baseline: 3.7716x; 3.7716x over previous
import functools

import jax
import jax.numpy as jnp
from jax import lax
from jax.experimental import pallas as pl
from jax.experimental.pallas import tpu as pltpu
from jax.experimental.pallas import tpu_sc as plsc

N = 50000
E = 800000
D = 64

NC = 2
NS = 16
L = 16

HALF = 25088
NPAD = 2 * HALF
ROWS_PER_TILE = HALF // NS
WB = 112

GARB = HALF
ACC_ROWS = HALF + 256
ZCHUNK = ACC_ROWS // NS

CH = 128
NCH = 392
EPT = CH * NCH
E_PAD = EPT * NS
NR = E_PAD // CH


def _spmm_body(second_layer, *refs):
    if second_layer:
        (e3_hbm, xsrc_hbm, xadd_hbm, yadd_hbm, out_hbm,
         eb0, eb1, eb2, eb3, msg0, msg1, libuf, wb3,
         si0, si1, si2, si3, sg0, sg1, acc) = refs
        ebuf = [eb0, eb1, eb2, eb3]
        msg = [msg0, msg1]
        semi = [si0, si1, si2, si3]
        semg = [sg0, sg1]
    else:
        (e3_hbm, xsrc_hbm, out_hbm,
         eb0, eb1, eb2, eb3, msg0, msg1, libuf,
         si0, si1, si2, si3, sg0, sg1, acc) = refs
        ebuf = [eb0, eb1, eb2, eb3]
        msg = [msg0, msg1]
        semi = [si0, si1, si2, si3]
        semg = [sg0, sg1]

    c = lax.axis_index("c")
    s = lax.axis_index("s")
    base_row = c * HALF
    iota = lax.iota(jnp.int32, L)
    zeros = jnp.zeros((L,), jnp.float32)
    ebase = s * NCH

    def _zero_msg(r, _):
        for j in range(4):
            msg0[r, pl.ds(j * L, L)] = zeros
        return 0
    lax.fori_loop(0, CH, _zero_msg, 0)
    zbase = s * ZCHUNK
    off = 0
    while off < ZCHUNK:
        sz = min(CH, ZCHUNK - off)
        pltpu.sync_copy(msg0.at[pl.ds(0, sz)], acc.at[pl.ds(zbase + off, sz)])
        off += sz

    pltpu.sync_copy(e3_hbm.at[ebase + 0], ebuf[0])
    pltpu.sync_copy(e3_hbm.at[ebase + 1], ebuf[1])
    for b in range(2):
        pltpu.async_copy(xsrc_hbm.at[ebuf[b].at[1]], msg[b], semg[b])
    pltpu.async_copy(e3_hbm.at[ebase + 2], ebuf[2], semi[2])
    pltpu.async_copy(e3_hbm.at[ebase + 3], ebuf[3], semi[3])

    plsc.subcore_barrier()

    def _quad(k, _):
        for b in range(4):
            ch = 4 * k + b
            m = b % 2
            pltpu.make_async_copy(
                xsrc_hbm.at[ebuf[b].at[1]], msg[m], semg[m]).wait()
            for g in range(CH // L):
                r = ebuf[b][0, pl.ds(g * L, L)]
                li = r - base_row
                oob = (li < 0) | (li >= HALF)
                garb = GARB + (b % 2) * 128 + g * L + iota
                libuf[0, pl.ds(g * L, L)] = jnp.where(oob, garb, li)
            def _scale(l, _, b=b, m=m):
                iv = ebuf[b][2, pl.ds(l * L, L)]
                vv = plsc.bitcast(iv, jnp.float32)
                for i in range(L):
                    v = vv[i]
                    row = l * L + i
                    for j in range(4):
                        sl = pl.ds(j * L, L)
                        msg[m][row, sl] = msg[m][row, sl] * v
                return 0
            lax.fori_loop(0, CH // L, _scale, 0)
            pltpu.sync_copy(msg[m], acc.at[libuf.at[0]], add=True)
            @pl.when(ch + 4 < NCH)
            def _():
                pltpu.async_copy(e3_hbm.at[ebase + ch + 4], ebuf[b], semi[b])
            @pl.when(ch + 2 < NCH)
            def _():
                b2 = (b + 2) % 4
                pltpu.make_async_copy(
                    e3_hbm.at[ebase], ebuf[b2], semi[b2]).wait()
                pltpu.async_copy(
                    xsrc_hbm.at[ebuf[b2].at[1]], msg[m], semg[m])
        return 0

    lax.fori_loop(0, NCH // 4, _quad, 0)
    plsc.subcore_barrier()

    if second_layer:
        def _wb(k, _):
            loc = s * ROWS_PER_TILE + k * WB
            gbase = base_row + loc
            pltpu.sync_copy(acc.at[pl.ds(loc, WB)], msg0.at[pl.ds(0, WB)])
            pltpu.sync_copy(xadd_hbm.at[pl.ds(gbase, WB)],
                            msg1.at[pl.ds(0, WB)])
            pltpu.sync_copy(yadd_hbm.at[pl.ds(gbase, WB)], wb3)

            def _add(r, _):
                for j in range(4):
                    sl = pl.ds(j * L, L)
                    msg0[r, sl] = msg0[r, sl] + msg1[r, sl] + wb3[r, sl]
                return 0
            lax.fori_loop(0, WB, _add, 0)
            pltpu.sync_copy(msg0.at[pl.ds(0, WB)], out_hbm.at[pl.ds(gbase, WB)])
            return 0
        lax.fori_loop(0, ROWS_PER_TILE // WB, _wb, 0)
    else:
        loc = s * ROWS_PER_TILE
        pltpu.sync_copy(acc.at[pl.ds(loc, ROWS_PER_TILE)],
                        out_hbm.at[pl.ds(base_row + loc, ROWS_PER_TILE)])


def _make_spmm(second_layer):
    scratch = [
        pltpu.VMEM((3, CH), jnp.int32),
        pltpu.VMEM((3, CH), jnp.int32),
        pltpu.VMEM((3, CH), jnp.int32),
        pltpu.VMEM((3, CH), jnp.int32),
        pltpu.VMEM((CH, D), jnp.float32),
        pltpu.VMEM((CH, D), jnp.float32),
        pltpu.VMEM((1, CH), jnp.int32),
    ]
    if second_layer:
        scratch += [pltpu.VMEM((WB, D), jnp.float32)]
    scratch += [pltpu.SemaphoreType.DMA] * 6
    scratch += [pltpu.VMEM_SHARED((ACC_ROWS, D), jnp.float32)]
    mesh = plsc.VectorSubcoreMesh(
        core_axis_name="c", subcore_axis_name="s",
        num_cores=NC, num_subcores=NS)
    return pl.kernel(
        functools.partial(_spmm_body, second_layer),
        out_type=jax.ShapeDtypeStruct((NPAD, D), jnp.float32),
        mesh=mesh,
        scratch_types=scratch,
        compiler_params=pltpu.CompilerParams(
            use_tc_tiling_on_sc=False, needs_layout_passes=False),
        name="spmm_layer2" if second_layer else "spmm_layer1",
    )


@jax.jit
def kernel(edge_index, adj_values, uEmbeds):
    pad = E_PAD - E
    fill = (jnp.arange(pad, dtype=jnp.int32) * 61) % N
    rows = jnp.concatenate([edge_index[0].astype(jnp.int32), fill])
    cols = jnp.concatenate([edge_index[1].astype(jnp.int32), fill])
    vals = jnp.concatenate([adj_values, jnp.zeros((pad,), jnp.float32)])
    vbits = lax.bitcast_convert_type(vals, jnp.int32)
    e3 = jnp.stack(
        [rows.reshape(NR, CH), cols.reshape(NR, CH), vbits.reshape(NR, CH)],
        axis=1)
    x_pad = jnp.pad(uEmbeds, ((0, NPAD - N), (0, 0)))

    y1 = _make_spmm(False)(e3, x_pad)
    out = _make_spmm(True)(e3, y1, x_pad, y1)
    return out[:N]


# speedup vs baseline: 9.0429x; 2.3976x over previous
import functools

import jax
import jax.numpy as jnp
from jax import lax
from jax.experimental import pallas as pl
from jax.experimental.pallas import tpu as pltpu
from jax.experimental.pallas import tpu_sc as plsc

N = 50000
E = 800000
D = 64

NC = 2
NS = 16
L = 16

HALF = 25088
NPAD = 2 * HALF
ROWS_PER_TILE = HALF // NS
WB = 112

GARB = HALF
ACC_ROWS = HALF + 256
ZCHUNK = ACC_ROWS // NS

CH = 128
NCH = 392
EPT = CH * NCH
E_PAD = EPT * NS
NR = E_PAD // CH


def _spmm_body(second_layer, *refs):
    if second_layer:
        (e3_hbm, xsrc_hbm, xadd_hbm, yadd_hbm, out_hbm,
         eb0, eb1, eb2, eb3, msg0, msg1, libuf, wb3,
         si0, si1, si2, si3, sg0, sg1, acc) = refs
        ebuf = [eb0, eb1, eb2, eb3]
        msg = [msg0, msg1]
        semi = [si0, si1, si2, si3]
        semg = [sg0, sg1]
    else:
        (e3_hbm, xsrc_hbm, out_hbm,
         eb0, eb1, eb2, eb3, msg0, msg1, libuf,
         si0, si1, si2, si3, sg0, sg1, acc) = refs
        ebuf = [eb0, eb1, eb2, eb3]
        msg = [msg0, msg1]
        semi = [si0, si1, si2, si3]
        semg = [sg0, sg1]

    c = lax.axis_index("c")
    s = lax.axis_index("s")
    base_row = c * HALF
    iota = lax.iota(jnp.int32, L)
    zeros = jnp.zeros((L,), jnp.float32)
    ebase = s * NCH

    def _zero_msg(r, _):
        for j in range(4):
            msg0[r, pl.ds(j * L, L)] = zeros
        return 0
    lax.fori_loop(0, CH, _zero_msg, 0)
    zbase = s * ZCHUNK
    off = 0
    while off < ZCHUNK:
        sz = min(CH, ZCHUNK - off)
        pltpu.sync_copy(msg0.at[pl.ds(0, sz)], acc.at[pl.ds(zbase + off, sz)])
        off += sz

    pltpu.sync_copy(e3_hbm.at[ebase + 0], ebuf[0])
    pltpu.sync_copy(e3_hbm.at[ebase + 1], ebuf[1])
    for b in range(2):
        pltpu.async_copy(xsrc_hbm.at[ebuf[b].at[1]], msg[b], semg[b])
    pltpu.async_copy(e3_hbm.at[ebase + 2], ebuf[2], semi[2])
    pltpu.async_copy(e3_hbm.at[ebase + 3], ebuf[3], semi[3])

    plsc.subcore_barrier()

    def _quad(k, _):
        for b in range(4):
            ch = 4 * k + b
            m = b % 2
            pltpu.make_async_copy(
                xsrc_hbm.at[ebuf[b].at[1]], msg[m], semg[m]).wait()
            for g in range(CH // L):
                r = ebuf[b][0, pl.ds(g * L, L)]
                li = r - base_row
                oob = (li < 0) | (li >= HALF)
                garb = GARB + (b % 2) * 128 + g * L + iota
                libuf[0, pl.ds(g * L, L)] = jnp.where(oob, garb, li)
            @plsc.parallel_loop(0, CH // L)
            def _scale(l, b=b, m=m):
                iv = ebuf[b][2, pl.ds(l * L, L)]
                vv = plsc.bitcast(iv, jnp.float32)
                base = l * L
                for i0 in range(0, L, 4):
                    rows = [
                        [msg[m][base + i0 + i, pl.ds(j * L, L)]
                         for j in range(4)]
                        for i in range(4)
                    ]
                    for i in range(4):
                        v = vv[i0 + i]
                        for j in range(4):
                            sl = pl.ds(j * L, L)
                            msg[m][base + i0 + i, sl] = rows[i][j] * v
            pltpu.sync_copy(msg[m], acc.at[libuf.at[0]], add=True)
            @pl.when(ch + 4 < NCH)
            def _():
                pltpu.async_copy(e3_hbm.at[ebase + ch + 4], ebuf[b], semi[b])
            @pl.when(ch + 2 < NCH)
            def _():
                b2 = (b + 2) % 4
                pltpu.make_async_copy(
                    e3_hbm.at[ebase], ebuf[b2], semi[b2]).wait()
                pltpu.async_copy(
                    xsrc_hbm.at[ebuf[b2].at[1]], msg[m], semg[m])
        return 0

    lax.fori_loop(0, NCH // 4, _quad, 0)
    plsc.subcore_barrier()

    if second_layer:
        def _wb(k, _):
            loc = s * ROWS_PER_TILE + k * WB
            gbase = base_row + loc
            pltpu.sync_copy(acc.at[pl.ds(loc, WB)], msg0.at[pl.ds(0, WB)])
            pltpu.sync_copy(xadd_hbm.at[pl.ds(gbase, WB)],
                            msg1.at[pl.ds(0, WB)])
            pltpu.sync_copy(yadd_hbm.at[pl.ds(gbase, WB)], wb3)

            def _add(r, _):
                for j in range(4):
                    sl = pl.ds(j * L, L)
                    msg0[r, sl] = msg0[r, sl] + msg1[r, sl] + wb3[r, sl]
                return 0
            lax.fori_loop(0, WB, _add, 0)
            pltpu.sync_copy(msg0.at[pl.ds(0, WB)], out_hbm.at[pl.ds(gbase, WB)])
            return 0
        lax.fori_loop(0, ROWS_PER_TILE // WB, _wb, 0)
    else:
        loc = s * ROWS_PER_TILE
        pltpu.sync_copy(acc.at[pl.ds(loc, ROWS_PER_TILE)],
                        out_hbm.at[pl.ds(base_row + loc, ROWS_PER_TILE)])


def _make_spmm(second_layer):
    scratch = [
        pltpu.VMEM((3, CH), jnp.int32),
        pltpu.VMEM((3, CH), jnp.int32),
        pltpu.VMEM((3, CH), jnp.int32),
        pltpu.VMEM((3, CH), jnp.int32),
        pltpu.VMEM((CH, D), jnp.float32),
        pltpu.VMEM((CH, D), jnp.float32),
        pltpu.VMEM((1, CH), jnp.int32),
    ]
    if second_layer:
        scratch += [pltpu.VMEM((WB, D), jnp.float32)]
    scratch += [pltpu.SemaphoreType.DMA] * 6
    scratch += [pltpu.VMEM_SHARED((ACC_ROWS, D), jnp.float32)]
    mesh = plsc.VectorSubcoreMesh(
        core_axis_name="c", subcore_axis_name="s",
        num_cores=NC, num_subcores=NS)
    return pl.kernel(
        functools.partial(_spmm_body, second_layer),
        out_type=jax.ShapeDtypeStruct((NPAD, D), jnp.float32),
        mesh=mesh,
        scratch_types=scratch,
        compiler_params=pltpu.CompilerParams(
            use_tc_tiling_on_sc=False, needs_layout_passes=False),
        name="spmm_layer2" if second_layer else "spmm_layer1",
    )


@jax.jit
def kernel(edge_index, adj_values, uEmbeds):
    pad = E_PAD - E
    fill = (jnp.arange(pad, dtype=jnp.int32) * 61) % N
    rows = jnp.concatenate([edge_index[0].astype(jnp.int32), fill])
    cols = jnp.concatenate([edge_index[1].astype(jnp.int32), fill])
    vals = jnp.concatenate([adj_values, jnp.zeros((pad,), jnp.float32)])
    vbits = lax.bitcast_convert_type(vals, jnp.int32)
    e3 = jnp.stack(
        [rows.reshape(NR, CH), cols.reshape(NR, CH), vbits.reshape(NR, CH)],
        axis=1)
    x_pad = jnp.pad(uEmbeds, ((0, NPAD - N), (0, 0)))

    y1 = _make_spmm(False)(e3, x_pad)
    out = _make_spmm(True)(e3, y1, x_pad, y1)
    return out[:N]


# speedup vs baseline: 10.0758x; 1.1142x over previous
import functools

import jax
import jax.numpy as jnp
from jax import lax
from jax.experimental import pallas as pl
from jax.experimental.pallas import tpu as pltpu
from jax.experimental.pallas import tpu_sc as plsc

N = 50000
E = 800000
D = 64

NC = 2
NS = 16
L = 16

HALF = 25088
NPAD = 2 * HALF
ROWS_PER_TILE = HALF // NS
WB = 112

GARB = HALF
ACC_ROWS = HALF + 256
ZCHUNK = ACC_ROWS // NS

CH = 128
NCH = 393
EPT = CH * NCH
E_PAD = EPT * NS
NR = E_PAD // CH


def _spmm_body(second_layer, *refs):
    if second_layer:
        (e3_hbm, xsrc_hbm, xadd_hbm, yadd_hbm, out_hbm,
         eb0, eb1, eb2, msg0, msg1, msg2, li0, li1, li2,
         si0, si1, si2, sg0, sg1, sg2, ss0, ss1, ss2, acc) = refs
    else:
        (e3_hbm, xsrc_hbm, out_hbm,
         eb0, eb1, eb2, msg0, msg1, msg2, li0, li1, li2,
         si0, si1, si2, sg0, sg1, sg2, ss0, ss1, ss2, acc) = refs
    ebuf = [eb0, eb1, eb2]
    msg = [msg0, msg1, msg2]
    lib = [li0, li1, li2]
    semi = [si0, si1, si2]
    semg = [sg0, sg1, sg2]
    sems = [ss0, ss1, ss2]

    c = lax.axis_index("c")
    s = lax.axis_index("s")
    base_row = c * HALF
    iota = lax.iota(jnp.int32, L)
    zeros = jnp.zeros((L,), jnp.float32)
    ebase = s * NCH

    def _zero_msg(r, _):
        for j in range(4):
            msg0[r, pl.ds(j * L, L)] = zeros
        return 0
    lax.fori_loop(0, CH, _zero_msg, 0)
    zbase = s * ZCHUNK
    off = 0
    while off < ZCHUNK:
        sz = min(CH, ZCHUNK - off)
        pltpu.sync_copy(msg0.at[pl.ds(0, sz)], acc.at[pl.ds(zbase + off, sz)])
        off += sz

    pltpu.sync_copy(e3_hbm.at[ebase + 0], ebuf[0])
    pltpu.sync_copy(e3_hbm.at[ebase + 1], ebuf[1])
    for b in range(2):
        pltpu.async_copy(xsrc_hbm.at[ebuf[b].at[1]], msg[b], semg[b])
    pltpu.async_copy(e3_hbm.at[ebase + 2], ebuf[2], semi[2])

    plsc.subcore_barrier()

    def _tri(k, _):
        for b in range(3):
            ch = 3 * k + b
            pltpu.make_async_copy(
                xsrc_hbm.at[ebuf[b].at[1]], msg[b], semg[b]).wait()
            for g in range(CH // L):
                r = ebuf[b][0, pl.ds(g * L, L)]
                li = r - base_row
                oob = (li < 0) | (li >= HALF)
                garb = GARB + (b % 2) * 128 + g * L + iota
                lib[b][0, pl.ds(g * L, L)] = jnp.where(oob, garb, li)
            @plsc.parallel_loop(0, CH // L)
            def _scale(l, b=b):
                iv = ebuf[b][2, pl.ds(l * L, L)]
                vv = plsc.bitcast(iv, jnp.float32)
                base = l * L
                for i0 in range(0, L, 4):
                    rows = [
                        [msg[b][base + i0 + i, pl.ds(j * L, L)]
                         for j in range(4)]
                        for i in range(4)
                    ]
                    for i in range(4):
                        v = vv[i0 + i]
                        for j in range(4):
                            sl = pl.ds(j * L, L)
                            msg[b][base + i0 + i, sl] = rows[i][j] * v
            pltpu.async_copy(msg[b], acc.at[lib[b].at[0]], sems[b], add=True)
            @pl.when(ch + 3 < NCH)
            def _():
                pltpu.async_copy(e3_hbm.at[ebase + ch + 3], ebuf[b], semi[b])
            @pl.when(ch + 2 < NCH)
            def _():
                b2 = (b + 2) % 3
                pltpu.make_async_copy(
                    e3_hbm.at[ebase], ebuf[b2], semi[b2]).wait()

                @pl.when(ch >= 1)
                def _():
                    pltpu.make_async_copy(
                        msg[b2], acc.at[lib[b2].at[0]], sems[b2]).wait()
                pltpu.async_copy(
                    xsrc_hbm.at[ebuf[b2].at[1]], msg[b2], semg[b2])
        return 0

    lax.fori_loop(0, NCH // 3, _tri, 0)
    for q in range(3):
        pltpu.make_async_copy(msg[q], acc.at[lib[q].at[0]], sems[q]).wait()
    plsc.subcore_barrier()

    if second_layer:
        def _wb(k, _):
            loc = s * ROWS_PER_TILE + k * WB
            gbase = base_row + loc
            pltpu.sync_copy(acc.at[pl.ds(loc, WB)], msg0.at[pl.ds(0, WB)])
            pltpu.sync_copy(xadd_hbm.at[pl.ds(gbase, WB)],
                            msg1.at[pl.ds(0, WB)])
            pltpu.sync_copy(yadd_hbm.at[pl.ds(gbase, WB)],
                            msg2.at[pl.ds(0, WB)])

            @plsc.parallel_loop(0, WB)
            def _add(r):
                for j in range(4):
                    sl = pl.ds(j * L, L)
                    msg0[r, sl] = msg0[r, sl] + msg1[r, sl] + msg2[r, sl]
            pltpu.sync_copy(msg0.at[pl.ds(0, WB)], out_hbm.at[pl.ds(gbase, WB)])
            return 0
        lax.fori_loop(0, ROWS_PER_TILE // WB, _wb, 0)
    else:
        loc = s * ROWS_PER_TILE
        pltpu.sync_copy(acc.at[pl.ds(loc, ROWS_PER_TILE)],
                        out_hbm.at[pl.ds(base_row + loc, ROWS_PER_TILE)])


def _make_spmm(second_layer):
    scratch = [
        pltpu.VMEM((3, CH), jnp.int32),
        pltpu.VMEM((3, CH), jnp.int32),
        pltpu.VMEM((3, CH), jnp.int32),
        pltpu.VMEM((CH, D), jnp.float32),
        pltpu.VMEM((CH, D), jnp.float32),
        pltpu.VMEM((CH, D), jnp.float32),
        pltpu.VMEM((1, CH), jnp.int32),
        pltpu.VMEM((1, CH), jnp.int32),
        pltpu.VMEM((1, CH), jnp.int32),
    ]
    scratch += [pltpu.SemaphoreType.DMA] * 9
    scratch += [pltpu.VMEM_SHARED((ACC_ROWS, D), jnp.float32)]
    mesh = plsc.VectorSubcoreMesh(
        core_axis_name="c", subcore_axis_name="s",
        num_cores=NC, num_subcores=NS)
    return pl.kernel(
        functools.partial(_spmm_body, second_layer),
        out_type=jax.ShapeDtypeStruct((NPAD, D), jnp.float32),
        mesh=mesh,
        scratch_types=scratch,
        compiler_params=pltpu.CompilerParams(
            use_tc_tiling_on_sc=False, needs_layout_passes=False),
        name="spmm_layer2" if second_layer else "spmm_layer1",
    )


@jax.jit
def kernel(edge_index, adj_values, uEmbeds):
    pad = E_PAD - E
    fill = (jnp.arange(pad, dtype=jnp.int32) * 61) % N
    rows = jnp.concatenate([edge_index[0].astype(jnp.int32), fill])
    cols = jnp.concatenate([edge_index[1].astype(jnp.int32), fill])
    vals = jnp.concatenate([adj_values, jnp.zeros((pad,), jnp.float32)])
    vbits = lax.bitcast_convert_type(vals, jnp.int32)
    e3 = jnp.stack(
        [rows.reshape(NR, CH), cols.reshape(NR, CH), vbits.reshape(NR, CH)],
        axis=1)
    x_pad = jnp.pad(uEmbeds, ((0, NPAD - N), (0, 0)))

    y1 = _make_spmm(False)(e3, x_pad)
    out = _make_spmm(True)(e3, y1, x_pad, y1)
    return out[:N]


# speedup vs baseline: 14.1631x; 1.4057x over previous
import functools

import jax
import jax.numpy as jnp
from jax import lax
from jax.experimental import pallas as pl
from jax.experimental.pallas import tpu as pltpu
from jax.experimental.pallas import tpu_sc as plsc

N = 50000
E = 800000
D = 64

NC = 2
NS = 16
L = 16

HALF = 25088
NPAD = 2 * HALF
ROWS_PER_TILE = HALF // NS
WB = 112

GARB = HALF
ACC_ROWS = HALF + 256
ZCHUNK = ACC_ROWS // NS

CH = 128
PCH_SRC = 198
E_PAD = 32 * PCH_SRC * CH
NR = E_PAD // CH
PCH = PCH_SRC + 2
KMAX = (2 * PCH + 4) // 3


def _partition_body(e3_hbm, part_hbm, cnt_hbm,
                    eb0, eb1, eb2, pbuf, cbuf,
                    si0, si1, si2, sf0, sf1):
    ebuf = [eb0, eb1, eb2]
    semi = [si0, si1, si2]
    semf = [sf0, sf1]

    c = lax.axis_index("c")
    s = lax.axis_index("s")
    w = c * NS + s
    iota = lax.iota(jnp.int32, L)
    zeros_i = jnp.zeros((L,), jnp.int32)
    ebase = w * PCH_SRC

    pltpu.sync_copy(e3_hbm.at[ebase + 0], ebuf[0])
    pltpu.sync_copy(e3_hbm.at[ebase + 1], ebuf[1])
    pltpu.sync_copy(e3_hbm.at[ebase + 2], ebuf[2])

    def _scatter_triple(h_vec, q, lane, li, col, vb, m):
        plsc.store_scatter(pbuf, [h_vec, q, zeros_i, lane], li, mask=m)
        plsc.store_scatter(pbuf, [h_vec, q, zeros_i + 1, lane], col, mask=m)
        plsc.store_scatter(pbuf, [h_vec, q, zeros_i + 2, lane], vb, mask=m)

    def _flush(h, fid):
        @pl.when(fid >= 4)
        def _():
            pltpu.make_async_copy(
                pbuf.at[h, 0], part_hbm.at[h, w, 0], semf[h]).wait()
        pltpu.async_copy(
            pbuf.at[h, fid & 3], part_hbm.at[h, w, fid], semf[h])

    def _tri(k, carry):
        p = list(carry)
        for b in range(3):
            ch = 3 * k + b

            @pl.when(k >= 1)
            def _():
                pltpu.make_async_copy(
                    e3_hbm.at[ebase], ebuf[b], semi[b]).wait()
            for g in range(CH // L):
                sl = pl.ds(g * L, L)
                r = ebuf[b][0, sl]
                col = ebuf[b][1, sl]
                vb = ebuf[b][2, sl]
                m1 = r >= HALF
                cum1 = plsc.cumsum(jnp.where(m1, 1, 0))
                cnt1 = cum1[L - 1]
                li = jnp.where(m1, r - HALF, r)
                pos = [p[0] + iota - cum1, p[1] + cum1 - 1]
                for h in range(2):
                    slot = pos[h] & 511
                    q = lax.shift_right_logical(slot, 7)
                    lane = slot & 127
                    m = jnp.logical_not(m1) if h == 0 else m1
                    _scatter_triple(zeros_i + h, q, lane, li, col, vb, m)
                pnew = [p[0] + (L - cnt1), p[1] + cnt1]
                for h in range(2):
                    oc = lax.shift_right_logical(p[h], 7)
                    ncc = lax.shift_right_logical(pnew[h], 7)

                    @pl.when(ncc > oc)
                    def _(h=h, oc=oc):
                        _flush(h, oc)
                p = pnew

            @pl.when(k < PCH_SRC // 3 - 1)
            def _():
                pltpu.async_copy(e3_hbm.at[ebase + ch + 3], ebuf[b], semi[b])
        return tuple(p)

    p0, p1 = lax.fori_loop(0, PCH_SRC // 3, _tri,
                           (jnp.int32(0), jnp.int32(0)))

    cnts = []
    for h, p in ((0, p0), (1, p1)):
        cnts.append(jnp.maximum(lax.shift_right_logical(p + 127, 7), 2))
        for sg in range(16):
            pos = p + sg * L + iota
            slot = pos & 511
            q = lax.shift_right_logical(slot, 7)
            lane = slot & 127
            li = GARB + sg * L + iota
            _scatter_triple(zeros_i + h, q, lane, li, iota * 16 + sg,
                            zeros_i, None)
        base_fid = lax.shift_right_logical(p, 7)
        for t in range(2):
            _flush(h, base_fid + t)
        nwait = jnp.minimum(base_fid + 2, 4)

        def _drain(t, _, h=h):
            pltpu.make_async_copy(
                pbuf.at[h, 0], part_hbm.at[h, w, 0], semf[h]).wait()
            return 0
        lax.fori_loop(0, nwait, _drain, 0)

    cvec = jnp.where(iota == 0, cnts[0], jnp.where(iota == 1, cnts[1], 0))
    cbuf[0, pl.ds(0, L)] = cvec
    pltpu.sync_copy(cbuf, cnt_hbm.at[pl.ds(w, 1)])


def _make_partition():
    scratch = [
        pltpu.VMEM((3, CH), jnp.int32),
        pltpu.VMEM((3, CH), jnp.int32),
        pltpu.VMEM((3, CH), jnp.int32),
        pltpu.VMEM((2, 4, 3, CH), jnp.int32),
        pltpu.VMEM((1, L), jnp.int32),
    ]
    scratch += [pltpu.SemaphoreType.DMA] * 5
    mesh = plsc.VectorSubcoreMesh(
        core_axis_name="c", subcore_axis_name="s",
        num_cores=NC, num_subcores=NS)
    return pl.kernel(
        _partition_body,
        out_type=(
            jax.ShapeDtypeStruct((2, 32, PCH, 3, CH), jnp.int32),
            jax.ShapeDtypeStruct((32, L), jnp.int32),
        ),
        mesh=mesh,
        scratch_types=scratch,
        compiler_params=pltpu.CompilerParams(
            use_tc_tiling_on_sc=False, needs_layout_passes=False),
        name="edge_partition",
    )


def _spmm_body(second_layer, *refs):
    if second_layer:
        (part_hbm, cnt_hbm, xsrc_hbm, xadd_hbm, yadd_hbm, out_hbm,
         eb0, eb1, eb2, msg0, msg1, msg2, li0, li1, li2, cbuf,
         si0, si1, si2, sg0, sg1, sg2, ss0, ss1, ss2, acc) = refs
    else:
        (part_hbm, cnt_hbm, xsrc_hbm, out_hbm,
         eb0, eb1, eb2, msg0, msg1, msg2, li0, li1, li2, cbuf,
         si0, si1, si2, sg0, sg1, sg2, ss0, ss1, ss2, acc) = refs
    ebuf = [eb0, eb1, eb2]
    msg = [msg0, msg1, msg2]
    lib = [li0, li1, li2]
    semi = [si0, si1, si2]
    semg = [sg0, sg1, sg2]
    sems = [ss0, ss1, ss2]

    c = lax.axis_index("c")
    s = lax.axis_index("s")
    base_row = c * HALF
    zeros = jnp.zeros((L,), jnp.float32)

    pltpu.sync_copy(cnt_hbm.at[pl.ds(2 * s, 2)], cbuf)
    va = cbuf[0, pl.ds(0, L)]
    vb = cbuf[1, pl.ds(0, L)]
    n0 = jnp.where(c == 0, va[0], va[1])
    n1 = jnp.where(c == 0, vb[0], vb[1])
    nn = n0 + n1

    def echunk(j):
        w = jnp.where(j < n0, 2 * s, 2 * s + 1)
        jj = jnp.where(j < n0, j, j - n0)
        return part_hbm.at[c, w, jj]

    def _zero_msg(r, _):
        for j in range(4):
            msg0[r, pl.ds(j * L, L)] = zeros
        return 0
    lax.fori_loop(0, CH, _zero_msg, 0)
    zbase = s * ZCHUNK
    off = 0
    while off < ZCHUNK:
        sz = min(CH, ZCHUNK - off)
        pltpu.sync_copy(msg0.at[pl.ds(0, sz)], acc.at[pl.ds(zbase + off, sz)])
        off += sz

    pltpu.sync_copy(echunk(0), ebuf[0])
    pltpu.sync_copy(echunk(1), ebuf[1])
    for b in range(2):
        pltpu.async_copy(xsrc_hbm.at[ebuf[b].at[1]], msg[b], semg[b])
    pltpu.async_copy(echunk(2), ebuf[2], semi[2])

    plsc.subcore_barrier()

    def _tri(k, _):
        for b in range(3):
            ch = 3 * k + b

            @pl.when(ch < nn)
            def _():
                pltpu.make_async_copy(
                    xsrc_hbm.at[ebuf[b].at[1]], msg[b], semg[b]).wait()
                for g in range(CH // L):
                    sl = pl.ds(g * L, L)
                    lib[b][0, sl] = ebuf[b][0, sl]
                @plsc.parallel_loop(0, CH // L)
                def _scale(l, b=b):
                    iv = ebuf[b][2, pl.ds(l * L, L)]
                    vv = plsc.bitcast(iv, jnp.float32)
                    base = l * L
                    for i0 in range(0, L, 4):
                        rows = [
                            [msg[b][base + i0 + i, pl.ds(j * L, L)]
                             for j in range(4)]
                            for i in range(4)
                        ]
                        for i in range(4):
                            v = vv[i0 + i]
                            for j in range(4):
                                sl = pl.ds(j * L, L)
                                msg[b][base + i0 + i, sl] = rows[i][j] * v
                pltpu.async_copy(msg[b], acc.at[lib[b].at[0]], sems[b],
                                 add=True)

            @pl.when(ch + 3 < nn)
            def _():
                pltpu.async_copy(echunk(ch + 3), ebuf[b], semi[b])
            @pl.when(ch + 2 < nn)
            def _():
                b2 = (b + 2) % 3
                pltpu.make_async_copy(
                    part_hbm.at[0, 0, 0], ebuf[b2], semi[b2]).wait()

                @pl.when(ch >= 1)
                def _():
                    pltpu.make_async_copy(
                        msg[b2], acc.at[lib[b2].at[0]], sems[b2]).wait()
                pltpu.async_copy(
                    xsrc_hbm.at[ebuf[b2].at[1]], msg[b2], semg[b2])
        return 0

    lax.fori_loop(0, KMAX, _tri, 0)
    for q in range(3):
        pltpu.make_async_copy(msg[q], acc.at[lib[q].at[0]], sems[q]).wait()
    plsc.subcore_barrier()

    if second_layer:
        def _wb(k, _):
            loc = s * ROWS_PER_TILE + k * WB
            gbase = base_row + loc
            pltpu.sync_copy(acc.at[pl.ds(loc, WB)], msg0.at[pl.ds(0, WB)])
            pltpu.sync_copy(xadd_hbm.at[pl.ds(gbase, WB)],
                            msg1.at[pl.ds(0, WB)])
            pltpu.sync_copy(yadd_hbm.at[pl.ds(gbase, WB)],
                            msg2.at[pl.ds(0, WB)])

            @plsc.parallel_loop(0, WB)
            def _add(r):
                for j in range(4):
                    sl = pl.ds(j * L, L)
                    msg0[r, sl] = msg0[r, sl] + msg1[r, sl] + msg2[r, sl]
            pltpu.sync_copy(msg0.at[pl.ds(0, WB)], out_hbm.at[pl.ds(gbase, WB)])
            return 0
        lax.fori_loop(0, ROWS_PER_TILE // WB, _wb, 0)
    else:
        loc = s * ROWS_PER_TILE
        pltpu.sync_copy(acc.at[pl.ds(loc, ROWS_PER_TILE)],
                        out_hbm.at[pl.ds(base_row + loc, ROWS_PER_TILE)])


def _make_spmm(second_layer):
    scratch = [
        pltpu.VMEM((3, CH), jnp.int32),
        pltpu.VMEM((3, CH), jnp.int32),
        pltpu.VMEM((3, CH), jnp.int32),
        pltpu.VMEM((CH, D), jnp.float32),
        pltpu.VMEM((CH, D), jnp.float32),
        pltpu.VMEM((CH, D), jnp.float32),
        pltpu.VMEM((1, CH), jnp.int32),
        pltpu.VMEM((1, CH), jnp.int32),
        pltpu.VMEM((1, CH), jnp.int32),
        pltpu.VMEM((2, L), jnp.int32),
    ]
    scratch += [pltpu.SemaphoreType.DMA] * 9
    scratch += [pltpu.VMEM_SHARED((ACC_ROWS, D), jnp.float32)]
    mesh = plsc.VectorSubcoreMesh(
        core_axis_name="c", subcore_axis_name="s",
        num_cores=NC, num_subcores=NS)
    return pl.kernel(
        functools.partial(_spmm_body, second_layer),
        out_type=jax.ShapeDtypeStruct((NPAD, D), jnp.float32),
        mesh=mesh,
        scratch_types=scratch,
        compiler_params=pltpu.CompilerParams(
            use_tc_tiling_on_sc=False, needs_layout_passes=False),
        name="spmm_layer2" if second_layer else "spmm_layer1",
    )


@jax.jit
def kernel(edge_index, adj_values, uEmbeds):
    pad = E_PAD - E
    fill = (jnp.arange(pad, dtype=jnp.int32) * 61) % N
    rows = jnp.concatenate([edge_index[0].astype(jnp.int32), fill])
    cols = jnp.concatenate([edge_index[1].astype(jnp.int32), fill])
    vals = jnp.concatenate([adj_values, jnp.zeros((pad,), jnp.float32)])
    vbits = lax.bitcast_convert_type(vals, jnp.int32)
    e3 = jnp.stack(
        [rows.reshape(NR, CH), cols.reshape(NR, CH), vbits.reshape(NR, CH)],
        axis=1)
    x_pad = jnp.pad(uEmbeds, ((0, NPAD - N), (0, 0)))

    part, cnt = _make_partition()(e3)
    y1 = _make_spmm(False)(part, cnt, x_pad)
    out = _make_spmm(True)(part, cnt, y1, x_pad, y1)
    return out[:N]


# speedup vs baseline: 15.2639x; 1.0777x over previous
import functools

import jax
import jax.numpy as jnp
from jax import lax
from jax.experimental import pallas as pl
from jax.experimental.pallas import tpu as pltpu
from jax.experimental.pallas import tpu_sc as plsc

N = 50000
E = 800000
D = 64

NC = 2
NS = 16
L = 16

HALF = 25088
NPAD = 2 * HALF
ROWS_PER_TILE = HALF // NS
WB = 112

GARB = HALF
ACC_ROWS = HALF + 256
ZCHUNK = ACC_ROWS // NS

CH = 128
EPT_SRC = E // 32
NCHP = 196
TAIL = EPT_SRC - 195 * CH
PCH = NCHP + 2
KMAX = (2 * PCH + 4) // 3


def _partition_body(ei_hbm, av_hbm, part_hbm, cnt_hbm,
                    eb0, eb1, eb2, vb0, vb1, vb2, pbuf, cbuf,
                    si0, si1, si2, sf0, sf1):
    ebuf = [eb0, eb1, eb2]
    vbuf = [vb0, vb1, vb2]
    semi = [si0, si1, si2]
    semf = [sf0, sf1]

    c = lax.axis_index("c")
    s = lax.axis_index("s")
    w = c * NS + s
    iota = lax.iota(jnp.int32, L)
    zeros_i = jnp.zeros((L,), jnp.int32)
    ebase = w * EPT_SRC

    def _load(ch, b, sync):
        base = ebase + ch * CH
        if sync:
            pltpu.sync_copy(ei_hbm.at[pl.ds(0, 1), pl.ds(base, CH)],
                            ebuf[b].at[pl.ds(0, 1)])
            pltpu.sync_copy(ei_hbm.at[pl.ds(1, 1), pl.ds(base, CH)],
                            ebuf[b].at[pl.ds(1, 1)])
            pltpu.sync_copy(av_hbm.at[pl.ds(base, CH)], vbuf[b])
        else:
            pltpu.async_copy(ei_hbm.at[pl.ds(0, 1), pl.ds(base, CH)],
                             ebuf[b].at[pl.ds(0, 1)], semi[b])
            pltpu.async_copy(ei_hbm.at[pl.ds(1, 1), pl.ds(base, CH)],
                             ebuf[b].at[pl.ds(1, 1)], semi[b])
            pltpu.async_copy(av_hbm.at[pl.ds(base, CH)], vbuf[b], semi[b])

    def _drain_load(b):
        for t in range(2):
            pltpu.make_async_copy(
                ei_hbm.at[pl.ds(0, 1), pl.ds(0, CH)],
                ebuf[b].at[pl.ds(t, 1)], semi[b]).wait()
        pltpu.make_async_copy(av_hbm.at[pl.ds(0, CH)], vbuf[b],
                              semi[b]).wait()

    for cc in range(3):
        _load(cc, cc, True)

    def _scatter_triple(h_vec, q, lane, li, col, vb, m):
        plsc.store_scatter(pbuf, [h_vec, q, zeros_i, lane], li, mask=m)
        plsc.store_scatter(pbuf, [h_vec, q, zeros_i + 1, lane], col, mask=m)
        plsc.store_scatter(pbuf, [h_vec, q, zeros_i + 2, lane], vb, mask=m)

    def _flush(h, fid):
        @pl.when(fid >= 4)
        def _():
            pltpu.make_async_copy(
                pbuf.at[h, 0], part_hbm.at[h, w, 0], semf[h]).wait()
        pltpu.async_copy(
            pbuf.at[h, fid & 3], part_hbm.at[h, w, fid], semf[h])

    def _group(b, g, p, valid):
        sl = pl.ds(g * L, L)
        r = ebuf[b][0, sl]
        col = ebuf[b][1, sl]
        vb = plsc.bitcast(vbuf[b][sl], jnp.int32)
        m1 = r >= HALF
        li = jnp.where(m1, r - HALF, r)
        if valid is None:
            cum1 = plsc.cumsum(jnp.where(m1, 1, 0))
            cnt1 = cum1[L - 1]
            masks = [jnp.logical_not(m1), m1]
            pos = [p[0] + iota - cum1, p[1] + cum1 - 1]
            cnts = [L - cnt1, cnt1]
        else:
            ok = iota < valid
            m1v = jnp.logical_and(m1, ok)
            m0v = jnp.logical_and(jnp.logical_not(m1), ok)
            cum1 = plsc.cumsum(jnp.where(m1v, 1, 0))
            cum0 = plsc.cumsum(jnp.where(m0v, 1, 0))
            masks = [m0v, m1v]
            pos = [p[0] + cum0 - 1, p[1] + cum1 - 1]
            cnts = [cum0[L - 1], cum1[L - 1]]
        pnew = [p[0] + cnts[0], p[1] + cnts[1]]
        for h in range(2):
            slot = pos[h] & 511
            q = lax.shift_right_logical(slot, 7)
            lane = slot & 127
            _scatter_triple(zeros_i + h, q, lane, li, col, vb, masks[h])
            oc = lax.shift_right_logical(p[h], 7)
            ncc = lax.shift_right_logical(pnew[h], 7)

            @pl.when(ncc > oc)
            def _(h=h, oc=oc):
                _flush(h, oc)
        return pnew

    def _tri(k, carry):
        p = list(carry)
        for b in range(3):
            ch = 3 * k + b

            @pl.when(k >= 1)
            def _():
                _drain_load(b)
            for g in range(CH // L):
                p = _group(b, g, p, None)

            @pl.when(k < 64)
            def _():
                _load(ch + 3, b, False)
        return tuple(p)

    p = lax.fori_loop(0, 65, _tri, (jnp.int32(0), jnp.int32(0)))
    p = list(p)

    _load(195, 0, True)
    for g in range(CH // L):
        nv = min(max(TAIL - g * L, 0), L)
        if nv == L:
            p = _group(0, g, p, None)
        elif nv > 0:
            p = _group(0, g, p, nv)
    p0, p1 = p

    cnts = []
    for h, p in ((0, p0), (1, p1)):
        cnts.append(jnp.maximum(lax.shift_right_logical(p + 127, 7), 2))
        for sg in range(16):
            pos = p + sg * L + iota
            slot = pos & 511
            q = lax.shift_right_logical(slot, 7)
            lane = slot & 127
            li = GARB + sg * L + iota
            _scatter_triple(zeros_i + h, q, lane, li, iota * 16 + sg,
                            zeros_i, None)
        base_fid = lax.shift_right_logical(p, 7)
        for t in range(2):
            _flush(h, base_fid + t)
        nwait = jnp.minimum(base_fid + 2, 4)

        def _drain(t, _, h=h):
            pltpu.make_async_copy(
                pbuf.at[h, 0], part_hbm.at[h, w, 0], semf[h]).wait()
            return 0
        lax.fori_loop(0, nwait, _drain, 0)

    cvec = jnp.where(iota == 0, cnts[0], jnp.where(iota == 1, cnts[1], 0))
    cbuf[0, pl.ds(0, L)] = cvec
    pltpu.sync_copy(cbuf, cnt_hbm.at[pl.ds(w, 1)])


def _make_partition():
    scratch = [
        pltpu.VMEM((3, CH), jnp.int32),
        pltpu.VMEM((3, CH), jnp.int32),
        pltpu.VMEM((3, CH), jnp.int32),
        pltpu.VMEM((CH,), jnp.float32),
        pltpu.VMEM((CH,), jnp.float32),
        pltpu.VMEM((CH,), jnp.float32),
        pltpu.VMEM((2, 4, 3, CH), jnp.int32),
        pltpu.VMEM((1, L), jnp.int32),
    ]
    scratch += [pltpu.SemaphoreType.DMA] * 5
    mesh = plsc.VectorSubcoreMesh(
        core_axis_name="c", subcore_axis_name="s",
        num_cores=NC, num_subcores=NS)
    return pl.kernel(
        _partition_body,
        out_type=(
            jax.ShapeDtypeStruct((2, 32, PCH, 3, CH), jnp.int32),
            jax.ShapeDtypeStruct((32, L), jnp.int32),
        ),
        mesh=mesh,
        scratch_types=scratch,
        compiler_params=pltpu.CompilerParams(
            use_tc_tiling_on_sc=False, needs_layout_passes=False),
        name="edge_partition",
    )


def _spmm_body(second_layer, *refs):
    if second_layer:
        (part_hbm, cnt_hbm, xsrc_hbm, xadd_hbm, yadd_hbm, out_hbm,
         eb0, eb1, eb2, msg0, msg1, msg2, li0, li1, li2, cbuf,
         si0, si1, si2, sg0, sg1, sg2, ss0, ss1, ss2, acc) = refs
    else:
        (part_hbm, cnt_hbm, xsrc_hbm, out_hbm,
         eb0, eb1, eb2, msg0, msg1, msg2, li0, li1, li2, cbuf,
         si0, si1, si2, sg0, sg1, sg2, ss0, ss1, ss2, acc) = refs
    ebuf = [eb0, eb1, eb2]
    msg = [msg0, msg1, msg2]
    lib = [li0, li1, li2]
    semi = [si0, si1, si2]
    semg = [sg0, sg1, sg2]
    sems = [ss0, ss1, ss2]

    c = lax.axis_index("c")
    s = lax.axis_index("s")
    base_row = c * HALF
    zeros = jnp.zeros((L,), jnp.float32)

    pltpu.sync_copy(cnt_hbm.at[pl.ds(2 * s, 2)], cbuf)
    va = cbuf[0, pl.ds(0, L)]
    vb = cbuf[1, pl.ds(0, L)]
    n0 = jnp.where(c == 0, va[0], va[1])
    n1 = jnp.where(c == 0, vb[0], vb[1])
    nn = n0 + n1

    def echunk(j):
        w = jnp.where(j < n0, 2 * s, 2 * s + 1)
        jj = jnp.where(j < n0, j, j - n0)
        return part_hbm.at[c, w, jj]

    def _zero_msg(r, _):
        for j in range(4):
            msg0[r, pl.ds(j * L, L)] = zeros
        return 0
    lax.fori_loop(0, CH, _zero_msg, 0)
    zbase = s * ZCHUNK
    off = 0
    while off < ZCHUNK:
        sz = min(CH, ZCHUNK - off)
        pltpu.sync_copy(msg0.at[pl.ds(0, sz)], acc.at[pl.ds(zbase + off, sz)])
        off += sz

    pltpu.sync_copy(echunk(0), ebuf[0])
    pltpu.sync_copy(echunk(1), ebuf[1])
    for b in range(2):
        pltpu.async_copy(xsrc_hbm.at[ebuf[b].at[1]], msg[b], semg[b])
    pltpu.async_copy(echunk(2), ebuf[2], semi[2])

    plsc.subcore_barrier()

    def _tri(k, _):
        for b in range(3):
            ch = 3 * k + b

            @pl.when(ch < nn)
            def _():
                pltpu.make_async_copy(
                    xsrc_hbm.at[ebuf[b].at[1]], msg[b], semg[b]).wait()
                for g in range(CH // L):
                    sl = pl.ds(g * L, L)
                    lib[b][0, sl] = ebuf[b][0, sl]
                @plsc.parallel_loop(0, CH // L)
                def _scale(l, b=b):
                    iv = ebuf[b][2, pl.ds(l * L, L)]
                    vv = plsc.bitcast(iv, jnp.float32)
                    base = l * L
                    for i0 in range(0, L, 4):
                        rows = [
                            [msg[b][base + i0 + i, pl.ds(j * L, L)]
                             for j in range(4)]
                            for i in range(4)
                        ]
                        for i in range(4):
                            v = vv[i0 + i]
                            for j in range(4):
                                sl = pl.ds(j * L, L)
                                msg[b][base + i0 + i, sl] = rows[i][j] * v
                pltpu.async_copy(msg[b], acc.at[lib[b].at[0]], sems[b],
                                 add=True)

            @pl.when(ch + 3 < nn)
            def _():
                pltpu.async_copy(echunk(ch + 3), ebuf[b], semi[b])
            @pl.when(ch + 2 < nn)
            def _():
                b2 = (b + 2) % 3
                pltpu.make_async_copy(
                    part_hbm.at[0, 0, 0], ebuf[b2], semi[b2]).wait()

                @pl.when(ch >= 1)
                def _():
                    pltpu.make_async_copy(
                        msg[b2], acc.at[lib[b2].at[0]], sems[b2]).wait()
                pltpu.async_copy(
                    xsrc_hbm.at[ebuf[b2].at[1]], msg[b2], semg[b2])
        return 0

    lax.fori_loop(0, KMAX, _tri, 0)
    for q in range(3):
        pltpu.make_async_copy(msg[q], acc.at[lib[q].at[0]], sems[q]).wait()
    plsc.subcore_barrier()

    if second_layer:
        def _wb(k, _):
            loc = s * ROWS_PER_TILE + k * WB
            gbase = base_row + loc
            pltpu.sync_copy(acc.at[pl.ds(loc, WB)], msg0.at[pl.ds(0, WB)])
            pltpu.sync_copy(xadd_hbm.at[pl.ds(gbase, WB)],
                            msg1.at[pl.ds(0, WB)])
            pltpu.sync_copy(yadd_hbm.at[pl.ds(gbase, WB)],
                            msg2.at[pl.ds(0, WB)])

            @plsc.parallel_loop(0, WB)
            def _add(r):
                for j in range(4):
                    sl = pl.ds(j * L, L)
                    msg0[r, sl] = msg0[r, sl] + msg1[r, sl] + msg2[r, sl]
            pltpu.sync_copy(msg0.at[pl.ds(0, WB)], out_hbm.at[pl.ds(gbase, WB)])
            return 0
        lax.fori_loop(0, ROWS_PER_TILE // WB, _wb, 0)
    else:
        loc = s * ROWS_PER_TILE
        pltpu.sync_copy(acc.at[pl.ds(loc, ROWS_PER_TILE)],
                        out_hbm.at[pl.ds(base_row + loc, ROWS_PER_TILE)])


def _make_spmm(second_layer):
    scratch = [
        pltpu.VMEM((3, CH), jnp.int32),
        pltpu.VMEM((3, CH), jnp.int32),
        pltpu.VMEM((3, CH), jnp.int32),
        pltpu.VMEM((CH, D), jnp.float32),
        pltpu.VMEM((CH, D), jnp.float32),
        pltpu.VMEM((CH, D), jnp.float32),
        pltpu.VMEM((1, CH), jnp.int32),
        pltpu.VMEM((1, CH), jnp.int32),
        pltpu.VMEM((1, CH), jnp.int32),
        pltpu.VMEM((2, L), jnp.int32),
    ]
    scratch += [pltpu.SemaphoreType.DMA] * 9
    scratch += [pltpu.VMEM_SHARED((ACC_ROWS, D), jnp.float32)]
    mesh = plsc.VectorSubcoreMesh(
        core_axis_name="c", subcore_axis_name="s",
        num_cores=NC, num_subcores=NS)
    return pl.kernel(
        functools.partial(_spmm_body, second_layer),
        out_type=jax.ShapeDtypeStruct((NPAD, D), jnp.float32),
        mesh=mesh,
        scratch_types=scratch,
        compiler_params=pltpu.CompilerParams(
            use_tc_tiling_on_sc=False, needs_layout_passes=False),
        name="spmm_layer2" if second_layer else "spmm_layer1",
    )


@jax.jit
def kernel(edge_index, adj_values, uEmbeds):
    ei = edge_index.astype(jnp.int32)
    x_pad = jnp.pad(uEmbeds, ((0, NPAD - N), (0, 0)))

    part, cnt = _make_partition()(ei, adj_values)
    y1 = _make_spmm(False)(part, cnt, uEmbeds)
    out = _make_spmm(True)(part, cnt, y1, x_pad, y1)
    return out[:N]


# speedup vs baseline: 15.2740x; 1.0007x over previous
import functools

import jax
import jax.numpy as jnp
from jax import lax
from jax.experimental import pallas as pl
from jax.experimental.pallas import tpu as pltpu
from jax.experimental.pallas import tpu_sc as plsc

N = 50000
E = 800000
D = 64

NC = 2
NS = 16
L = 16

HALF = 25088
NPAD = 2 * HALF
ROWS_PER_TILE = HALF // NS
WB = 112

GARB = HALF
ACC_ROWS = HALF + 256
ZCHUNK = ACC_ROWS // NS

CH = 128
EPT_SRC = E // 32
NCHP = 196
TAIL = EPT_SRC - 195 * CH
PCH = NCHP + 2
KMAX = (2 * PCH + 4) // 3


def _partition_body(ei_hbm, av_hbm, part_hbm, cnt_hbm,
                    eb0, eb1, eb2, vb0, vb1, vb2, pbuf, cbuf,
                    si0, si1, si2, sf0, sf1):
    ebuf = [eb0, eb1, eb2]
    vbuf = [vb0, vb1, vb2]
    semi = [si0, si1, si2]
    semf = [sf0, sf1]

    c = lax.axis_index("c")
    s = lax.axis_index("s")
    w = c * NS + s
    iota = lax.iota(jnp.int32, L)
    zeros_i = jnp.zeros((L,), jnp.int32)
    ebase = w * EPT_SRC

    def _load(ch, b, sync):
        base = ebase + ch * CH
        if sync:
            pltpu.sync_copy(ei_hbm.at[pl.ds(0, 1), pl.ds(base, CH)],
                            ebuf[b].at[pl.ds(0, 1)])
            pltpu.sync_copy(ei_hbm.at[pl.ds(1, 1), pl.ds(base, CH)],
                            ebuf[b].at[pl.ds(1, 1)])
            pltpu.sync_copy(av_hbm.at[pl.ds(base, CH)], vbuf[b])
        else:
            pltpu.async_copy(ei_hbm.at[pl.ds(0, 1), pl.ds(base, CH)],
                             ebuf[b].at[pl.ds(0, 1)], semi[b])
            pltpu.async_copy(ei_hbm.at[pl.ds(1, 1), pl.ds(base, CH)],
                             ebuf[b].at[pl.ds(1, 1)], semi[b])
            pltpu.async_copy(av_hbm.at[pl.ds(base, CH)], vbuf[b], semi[b])

    def _drain_load(b):
        for t in range(2):
            pltpu.make_async_copy(
                ei_hbm.at[pl.ds(0, 1), pl.ds(0, CH)],
                ebuf[b].at[pl.ds(t, 1)], semi[b]).wait()
        pltpu.make_async_copy(av_hbm.at[pl.ds(0, CH)], vbuf[b],
                              semi[b]).wait()

    for cc in range(3):
        _load(cc, cc, True)

    def _scatter_triple(h_vec, q, lane, li, col, vb, m):
        plsc.store_scatter(pbuf, [h_vec, q, zeros_i, lane], li, mask=m)
        plsc.store_scatter(pbuf, [h_vec, q, zeros_i + 1, lane], col, mask=m)
        plsc.store_scatter(pbuf, [h_vec, q, zeros_i + 2, lane], vb, mask=m)

    def _flush(h, fid):
        @pl.when(fid >= 4)
        def _():
            pltpu.make_async_copy(
                pbuf.at[h, 0], part_hbm.at[h, w, 0], semf[h]).wait()
        pltpu.async_copy(
            pbuf.at[h, fid & 3], part_hbm.at[h, w, fid], semf[h])

    def _group(b, g, p, valid):
        sl = pl.ds(g * L, L)
        r = ebuf[b][0, sl]
        col = ebuf[b][1, sl]
        vb = plsc.bitcast(vbuf[b][sl], jnp.int32)
        m1 = r >= HALF
        li = jnp.where(m1, r - HALF, r)
        if valid is None:
            cum1 = plsc.cumsum(jnp.where(m1, 1, 0))
            cnt1 = cum1[L - 1]
            masks = [jnp.logical_not(m1), m1]
            pos = [p[0] + iota - cum1, p[1] + cum1 - 1]
            cnts = [L - cnt1, cnt1]
        else:
            ok = iota < valid
            m1v = jnp.logical_and(m1, ok)
            m0v = jnp.logical_and(jnp.logical_not(m1), ok)
            cum1 = plsc.cumsum(jnp.where(m1v, 1, 0))
            cum0 = plsc.cumsum(jnp.where(m0v, 1, 0))
            masks = [m0v, m1v]
            pos = [p[0] + cum0 - 1, p[1] + cum1 - 1]
            cnts = [cum0[L - 1], cum1[L - 1]]
        pnew = [p[0] + cnts[0], p[1] + cnts[1]]
        for h in range(2):
            slot = pos[h] & 511
            q = lax.shift_right_logical(slot, 7)
            lane = slot & 127
            _scatter_triple(zeros_i + h, q, lane, li, col, vb, masks[h])
            oc = lax.shift_right_logical(p[h], 7)
            ncc = lax.shift_right_logical(pnew[h], 7)

            @pl.when(ncc > oc)
            def _(h=h, oc=oc):
                _flush(h, oc)
        return pnew

    def _tri(k, carry):
        p = list(carry)
        for b in range(3):
            ch = 3 * k + b

            @pl.when(k >= 1)
            def _():
                _drain_load(b)
            for g in range(CH // L):
                p = _group(b, g, p, None)

            @pl.when(k < 64)
            def _():
                _load(ch + 3, b, False)
        return tuple(p)

    p = lax.fori_loop(0, 65, _tri, (jnp.int32(0), jnp.int32(0)))
    p = list(p)

    _load(195, 0, True)
    for g in range(CH // L):
        nv = min(max(TAIL - g * L, 0), L)
        if nv == L:
            p = _group(0, g, p, None)
        elif nv > 0:
            p = _group(0, g, p, nv)
    p0, p1 = p

    cnts = []
    for h, p in ((0, p0), (1, p1)):
        cnts.append(jnp.maximum(lax.shift_right_logical(p + 127, 7), 2))
        for sg in range(16):
            pos = p + sg * L + iota
            slot = pos & 511
            q = lax.shift_right_logical(slot, 7)
            lane = slot & 127
            li = GARB + sg * L + iota
            _scatter_triple(zeros_i + h, q, lane, li, iota * 16 + sg,
                            zeros_i, None)
        base_fid = lax.shift_right_logical(p, 7)
        for t in range(2):
            _flush(h, base_fid + t)
        nwait = jnp.minimum(base_fid + 2, 4)

        def _drain(t, _, h=h):
            pltpu.make_async_copy(
                pbuf.at[h, 0], part_hbm.at[h, w, 0], semf[h]).wait()
            return 0
        lax.fori_loop(0, nwait, _drain, 0)

    cvec = jnp.where(iota == 0, cnts[0], jnp.where(iota == 1, cnts[1], 0))
    cbuf[0, pl.ds(0, L)] = cvec
    pltpu.sync_copy(cbuf, cnt_hbm.at[pl.ds(w, 1)])


def _make_partition():
    scratch = [
        pltpu.VMEM((3, CH), jnp.int32),
        pltpu.VMEM((3, CH), jnp.int32),
        pltpu.VMEM((3, CH), jnp.int32),
        pltpu.VMEM((CH,), jnp.float32),
        pltpu.VMEM((CH,), jnp.float32),
        pltpu.VMEM((CH,), jnp.float32),
        pltpu.VMEM((2, 4, 3, CH), jnp.int32),
        pltpu.VMEM((1, L), jnp.int32),
    ]
    scratch += [pltpu.SemaphoreType.DMA] * 5
    mesh = plsc.VectorSubcoreMesh(
        core_axis_name="c", subcore_axis_name="s",
        num_cores=NC, num_subcores=NS)
    return pl.kernel(
        _partition_body,
        out_type=(
            jax.ShapeDtypeStruct((2, 32, PCH, 3, CH), jnp.int32),
            jax.ShapeDtypeStruct((32, L), jnp.int32),
        ),
        mesh=mesh,
        scratch_types=scratch,
        compiler_params=pltpu.CompilerParams(
            use_tc_tiling_on_sc=False, needs_layout_passes=False),
        name="edge_partition",
    )


def _spmm_body(second_layer, *refs):
    if second_layer:
        (part_hbm, cnt_hbm, xsrc_hbm, xadd_hbm, yadd_hbm, out_hbm,
         eb0, eb1, eb2, msg0, msg1, msg2, li0, li1, li2, cbuf,
         si0, si1, si2, sg0, sg1, sg2, ss0, ss1, ss2, acc) = refs
    else:
        (part_hbm, cnt_hbm, xsrc_hbm, out_hbm,
         eb0, eb1, eb2, msg0, msg1, msg2, li0, li1, li2, cbuf,
         si0, si1, si2, sg0, sg1, sg2, ss0, ss1, ss2, acc) = refs
    ebuf = [eb0, eb1, eb2]
    msg = [msg0, msg1, msg2]
    lib = [li0, li1, li2]
    semi = [si0, si1, si2]
    semg = [sg0, sg1, sg2]
    sems = [ss0, ss1, ss2]

    c = lax.axis_index("c")
    s = lax.axis_index("s")
    base_row = c * HALF
    zeros = jnp.zeros((L,), jnp.float32)

    pltpu.sync_copy(cnt_hbm.at[pl.ds(2 * s, 2)], cbuf)
    va = cbuf[0, pl.ds(0, L)]
    vb = cbuf[1, pl.ds(0, L)]
    n0 = jnp.where(c == 0, va[0], va[1])
    n1 = jnp.where(c == 0, vb[0], vb[1])
    nn = n0 + n1

    def echunk(j):
        w = jnp.where(j < n0, 2 * s, 2 * s + 1)
        jj = jnp.where(j < n0, j, j - n0)
        return part_hbm.at[c, w, jj]

    def _zero_msg(r, _):
        for j in range(4):
            msg0[r, pl.ds(j * L, L)] = zeros
        return 0
    lax.fori_loop(0, CH, _zero_msg, 0)
    zbase = s * ZCHUNK
    off = 0
    while off < ZCHUNK:
        sz = min(CH, ZCHUNK - off)
        pltpu.sync_copy(msg0.at[pl.ds(0, sz)], acc.at[pl.ds(zbase + off, sz)])
        off += sz

    pltpu.sync_copy(echunk(0), ebuf[0])
    pltpu.sync_copy(echunk(1), ebuf[1])
    for b in range(2):
        pltpu.async_copy(xsrc_hbm.at[ebuf[b].at[1]], msg[b], semg[b])
    pltpu.async_copy(echunk(2), ebuf[2], semi[2])

    plsc.subcore_barrier()

    def _tri(k, _):
        for b in range(3):
            ch = 3 * k + b

            @pl.when(ch < nn)
            def _():
                pltpu.make_async_copy(
                    xsrc_hbm.at[ebuf[b].at[1]], msg[b], semg[b]).wait()
                for g in range(CH // L):
                    sl = pl.ds(g * L, L)
                    lib[b][0, sl] = ebuf[b][0, sl]
                @plsc.parallel_loop(0, CH // L)
                def _scale(l, b=b):
                    iv = ebuf[b][2, pl.ds(l * L, L)]
                    vv = plsc.bitcast(iv, jnp.float32)
                    base = l * L
                    for i0 in range(0, L, 4):
                        rows = [
                            [msg[b][base + i0 + i, pl.ds(j * L, L)]
                             for j in range(4)]
                            for i in range(4)
                        ]
                        for i in range(4):
                            v = vv[i0 + i]
                            for j in range(4):
                                sl = pl.ds(j * L, L)
                                msg[b][base + i0 + i, sl] = rows[i][j] * v
                pltpu.async_copy(msg[b], acc.at[lib[b].at[0]], sems[b],
                                 add=True)

            @pl.when(ch + 3 < nn)
            def _():
                pltpu.async_copy(echunk(ch + 3), ebuf[b], semi[b])
            @pl.when(ch + 2 < nn)
            def _():
                b2 = (b + 2) % 3
                pltpu.make_async_copy(
                    part_hbm.at[0, 0, 0], ebuf[b2], semi[b2]).wait()

                @pl.when(ch >= 1)
                def _():
                    pltpu.make_async_copy(
                        msg[b2], acc.at[lib[b2].at[0]], sems[b2]).wait()
                pltpu.async_copy(
                    xsrc_hbm.at[ebuf[b2].at[1]], msg[b2], semg[b2])
        return 0

    lax.fori_loop(0, KMAX, _tri, 0)
    for q in range(3):
        pltpu.make_async_copy(msg[q], acc.at[lib[q].at[0]], sems[q]).wait()
    plsc.subcore_barrier()

    if second_layer:
        def _wb(k, _):
            loc = s * ROWS_PER_TILE + k * WB
            gbase = base_row + loc
            pltpu.sync_copy(acc.at[pl.ds(loc, WB)], msg0.at[pl.ds(0, WB)])
            pltpu.sync_copy(xadd_hbm.at[pl.ds(gbase, WB)],
                            msg1.at[pl.ds(0, WB)])
            pltpu.sync_copy(yadd_hbm.at[pl.ds(gbase, WB)],
                            msg2.at[pl.ds(0, WB)])

            @plsc.parallel_loop(0, WB)
            def _add(r):
                for j in range(4):
                    sl = pl.ds(j * L, L)
                    msg0[r, sl] = msg0[r, sl] + msg1[r, sl] + msg2[r, sl]
            pltpu.sync_copy(msg0.at[pl.ds(0, WB)], out_hbm.at[pl.ds(gbase, WB)])
            return 0
        lax.fori_loop(0, ROWS_PER_TILE // WB, _wb, 0)
    else:
        loc = s * ROWS_PER_TILE
        pltpu.sync_copy(acc.at[pl.ds(loc, ROWS_PER_TILE)],
                        out_hbm.at[pl.ds(base_row + loc, ROWS_PER_TILE)])


def _make_spmm(second_layer):
    scratch = [
        pltpu.VMEM((3, CH), jnp.int32),
        pltpu.VMEM((3, CH), jnp.int32),
        pltpu.VMEM((3, CH), jnp.int32),
        pltpu.VMEM((CH, D), jnp.float32),
        pltpu.VMEM((CH, D), jnp.float32),
        pltpu.VMEM((CH, D), jnp.float32),
        pltpu.VMEM((1, CH), jnp.int32),
        pltpu.VMEM((1, CH), jnp.int32),
        pltpu.VMEM((1, CH), jnp.int32),
        pltpu.VMEM((2, L), jnp.int32),
    ]
    scratch += [pltpu.SemaphoreType.DMA] * 9
    scratch += [pltpu.VMEM_SHARED((ACC_ROWS, D), jnp.float32)]
    mesh = plsc.VectorSubcoreMesh(
        core_axis_name="c", subcore_axis_name="s",
        num_cores=NC, num_subcores=NS)
    return pl.kernel(
        functools.partial(_spmm_body, second_layer),
        out_type=jax.ShapeDtypeStruct((NPAD, D), jnp.float32),
        mesh=mesh,
        scratch_types=scratch,
        compiler_params=pltpu.CompilerParams(
            use_tc_tiling_on_sc=False, needs_layout_passes=False),
        name="spmm_layer2" if second_layer else "spmm_layer1",
    )


@jax.jit
def kernel(edge_index, adj_values, uEmbeds):
    ei = edge_index.astype(jnp.int32)
    x_pad = jnp.pad(uEmbeds, ((0, NPAD - N), (0, 0)))

    part, cnt = _make_partition()(ei, adj_values)
    y1 = _make_spmm(False)(part, cnt, x_pad)
    out = _make_spmm(True)(part, cnt, y1, x_pad, y1)
    return out[:N]


# speedup vs baseline: 15.4554x; 1.0119x over previous
import functools

import jax
import jax.numpy as jnp
from jax import lax
from jax.experimental import pallas as pl
from jax.experimental.pallas import tpu as pltpu
from jax.experimental.pallas import tpu_sc as plsc

N = 50000
E = 800000
D = 64

NC = 2
NS = 16
L = 16

HALF = 25088
NPAD = 2 * HALF
ROWS_PER_TILE = HALF // NS
WB = 112

GARB = HALF
ACC_ROWS = HALF + 256
ZCHUNK = ACC_ROWS // NS

CH = 128
EPT_SRC = E // 32
NCHP = 196
TAIL = EPT_SRC - 195 * CH
PCH = NCHP + 2
KMAX = (2 * PCH + 4) // 3


def _partition_body(ei_hbm, av_hbm, part_hbm, cnt_hbm,
                    eb0, eb1, eb2, vb0, vb1, vb2, pbuf, cbuf,
                    si0, si1, si2, sf0, sf1):
    ebuf = [eb0, eb1, eb2]
    vbuf = [vb0, vb1, vb2]
    semi = [si0, si1, si2]
    semf = [sf0, sf1]

    c = lax.axis_index("c")
    s = lax.axis_index("s")
    w = c * NS + s
    iota = lax.iota(jnp.int32, L)
    zeros_i = jnp.zeros((L,), jnp.int32)
    ebase = w * EPT_SRC

    def _load(ch, b, sync):
        base = ebase + ch * CH
        if sync:
            pltpu.sync_copy(ei_hbm.at[pl.ds(0, 1), pl.ds(base, CH)],
                            ebuf[b].at[pl.ds(0, 1)])
            pltpu.sync_copy(ei_hbm.at[pl.ds(1, 1), pl.ds(base, CH)],
                            ebuf[b].at[pl.ds(1, 1)])
            pltpu.sync_copy(av_hbm.at[pl.ds(base, CH)], vbuf[b])
        else:
            pltpu.async_copy(ei_hbm.at[pl.ds(0, 1), pl.ds(base, CH)],
                             ebuf[b].at[pl.ds(0, 1)], semi[b])
            pltpu.async_copy(ei_hbm.at[pl.ds(1, 1), pl.ds(base, CH)],
                             ebuf[b].at[pl.ds(1, 1)], semi[b])
            pltpu.async_copy(av_hbm.at[pl.ds(base, CH)], vbuf[b], semi[b])

    def _drain_load(b):
        for t in range(2):
            pltpu.make_async_copy(
                ei_hbm.at[pl.ds(0, 1), pl.ds(0, CH)],
                ebuf[b].at[pl.ds(t, 1)], semi[b]).wait()
        pltpu.make_async_copy(av_hbm.at[pl.ds(0, CH)], vbuf[b],
                              semi[b]).wait()

    for cc in range(3):
        _load(cc, cc, True)

    def _scatter_triple(h_vec, q, lane, li, col, vb, m):
        plsc.store_scatter(pbuf, [h_vec, q, zeros_i, lane], li, mask=m)
        plsc.store_scatter(pbuf, [h_vec, q, zeros_i + 1, lane], col, mask=m)
        plsc.store_scatter(pbuf, [h_vec, q, zeros_i + 2, lane], vb, mask=m)

    def _flush(h, fid):
        @pl.when(fid >= 4)
        def _():
            pltpu.make_async_copy(
                pbuf.at[h, 0], part_hbm.at[h, w, 0], semf[h]).wait()
        pltpu.async_copy(
            pbuf.at[h, fid & 3], part_hbm.at[h, w, fid], semf[h])

    def _group(b, g, p, valid):
        sl = pl.ds(g * L, L)
        r = ebuf[b][0, sl]
        col = ebuf[b][1, sl]
        vb = plsc.bitcast(vbuf[b][sl], jnp.int32)
        m1 = r >= HALF
        li = jnp.where(m1, r - HALF, r)
        if valid is None:
            cum1 = plsc.cumsum(jnp.where(m1, 1, 0))
            cnt1 = cum1[L - 1]
            masks = [jnp.logical_not(m1), m1]
            pos = [p[0] + iota - cum1, p[1] + cum1 - 1]
            cnts = [L - cnt1, cnt1]
        else:
            ok = iota < valid
            m1v = jnp.logical_and(m1, ok)
            m0v = jnp.logical_and(jnp.logical_not(m1), ok)
            cum1 = plsc.cumsum(jnp.where(m1v, 1, 0))
            cum0 = plsc.cumsum(jnp.where(m0v, 1, 0))
            masks = [m0v, m1v]
            pos = [p[0] + cum0 - 1, p[1] + cum1 - 1]
            cnts = [cum0[L - 1], cum1[L - 1]]
        pnew = [p[0] + cnts[0], p[1] + cnts[1]]
        for h in range(2):
            slot = pos[h] & 511
            q = lax.shift_right_logical(slot, 7)
            lane = slot & 127
            _scatter_triple(zeros_i + h, q, lane, li, col, vb, masks[h])
            oc = lax.shift_right_logical(p[h], 7)
            ncc = lax.shift_right_logical(pnew[h], 7)

            @pl.when(ncc > oc)
            def _(h=h, oc=oc):
                _flush(h, oc)
        return pnew

    def _tri(k, carry):
        p = list(carry)
        for b in range(3):
            ch = 3 * k + b

            @pl.when(k >= 1)
            def _():
                _drain_load(b)
            for g in range(CH // L):
                p = _group(b, g, p, None)

            @pl.when(k < 64)
            def _():
                _load(ch + 3, b, False)
        return tuple(p)

    p = lax.fori_loop(0, 65, _tri, (jnp.int32(0), jnp.int32(0)))
    p = list(p)

    _load(195, 0, True)
    for g in range(CH // L):
        nv = min(max(TAIL - g * L, 0), L)
        if nv == L:
            p = _group(0, g, p, None)
        elif nv > 0:
            p = _group(0, g, p, nv)
    p0, p1 = p

    cnts = []
    for h, p in ((0, p0), (1, p1)):
        cnts.append(jnp.maximum(lax.shift_right_logical(p + 127, 7), 2))
        for sg in range(16):
            pos = p + sg * L + iota
            slot = pos & 511
            q = lax.shift_right_logical(slot, 7)
            lane = slot & 127
            li = GARB + sg * L + iota
            _scatter_triple(zeros_i + h, q, lane, li, iota * 16 + sg,
                            zeros_i, None)
        base_fid = lax.shift_right_logical(p, 7)
        for t in range(2):
            _flush(h, base_fid + t)
        nwait = jnp.minimum(base_fid + 2, 4)

        def _drain(t, _, h=h):
            pltpu.make_async_copy(
                pbuf.at[h, 0], part_hbm.at[h, w, 0], semf[h]).wait()
            return 0
        lax.fori_loop(0, nwait, _drain, 0)

    cvec = jnp.where(iota == 0, cnts[0], jnp.where(iota == 1, cnts[1], 0))
    cbuf[0, pl.ds(0, L)] = cvec
    pltpu.sync_copy(cbuf, cnt_hbm.at[pl.ds(w, 1)])


def _make_partition():
    scratch = [
        pltpu.VMEM((3, CH), jnp.int32),
        pltpu.VMEM((3, CH), jnp.int32),
        pltpu.VMEM((3, CH), jnp.int32),
        pltpu.VMEM((CH,), jnp.float32),
        pltpu.VMEM((CH,), jnp.float32),
        pltpu.VMEM((CH,), jnp.float32),
        pltpu.VMEM((2, 4, 3, CH), jnp.int32),
        pltpu.VMEM((1, L), jnp.int32),
    ]
    scratch += [pltpu.SemaphoreType.DMA] * 5
    mesh = plsc.VectorSubcoreMesh(
        core_axis_name="c", subcore_axis_name="s",
        num_cores=NC, num_subcores=NS)
    return pl.kernel(
        _partition_body,
        out_type=(
            jax.ShapeDtypeStruct((2, 32, PCH, 3, CH), jnp.int32),
            jax.ShapeDtypeStruct((32, L), jnp.int32),
        ),
        mesh=mesh,
        scratch_types=scratch,
        compiler_params=pltpu.CompilerParams(
            use_tc_tiling_on_sc=False, needs_layout_passes=False),
        name="edge_partition",
    )


def _spmm_body(second_layer, *refs):
    if second_layer:
        (part_hbm, cnt_hbm, xsrc_hbm, xadd_hbm, yadd_hbm, out_hbm,
         eb0, eb1, eb2, msg0, msg1, msg2, li0, li1, li2, cbuf,
         si0, si1, si2, sg0, sg1, sg2, ss0, ss1, ss2, acc) = refs
    else:
        (part_hbm, cnt_hbm, xsrc_hbm, out_hbm,
         eb0, eb1, eb2, msg0, msg1, msg2, li0, li1, li2, cbuf,
         si0, si1, si2, sg0, sg1, sg2, ss0, ss1, ss2, acc) = refs
    ebuf = [eb0, eb1, eb2]
    msg = [msg0, msg1, msg2]
    lib = [li0, li1, li2]
    semi = [si0, si1, si2]
    semg = [sg0, sg1, sg2]
    sems = [ss0, ss1, ss2]

    c = lax.axis_index("c")
    s = lax.axis_index("s")
    base_row = c * HALF
    zeros = jnp.zeros((L,), jnp.float32)

    pltpu.sync_copy(cnt_hbm.at[pl.ds(2 * s, 2)], cbuf)
    va = cbuf[0, pl.ds(0, L)]
    vb = cbuf[1, pl.ds(0, L)]
    n0 = jnp.where(c == 0, va[0], va[1])
    n1 = jnp.where(c == 0, vb[0], vb[1])
    nn = n0 + n1

    def echunk(j):
        w = jnp.where(j < n0, 2 * s, 2 * s + 1)
        jj = jnp.where(j < n0, j, j - n0)
        return part_hbm.at[c, w, jj]

    def _zero_msg(r, _):
        for j in range(4):
            msg0[r, pl.ds(j * L, L)] = zeros
        return 0
    lax.fori_loop(0, CH, _zero_msg, 0)
    zbase = s * ZCHUNK
    zcopies = []
    off = 0
    while off < ZCHUNK:
        sz = min(CH, ZCHUNK - off)
        zcopies.append(pltpu.async_copy(
            msg0.at[pl.ds(0, sz)], acc.at[pl.ds(zbase + off, sz)], sg0))
        off += sz
    for h in zcopies:
        h.wait()

    pltpu.sync_copy(echunk(0), ebuf[0])
    pltpu.sync_copy(echunk(1), ebuf[1])
    for b in range(2):
        pltpu.async_copy(xsrc_hbm.at[ebuf[b].at[1]], msg[b], semg[b])
    pltpu.async_copy(echunk(2), ebuf[2], semi[2])

    plsc.subcore_barrier()

    def _tri(k, _):
        for b in range(3):
            ch = 3 * k + b

            @pl.when(ch < nn)
            def _():
                pltpu.make_async_copy(
                    xsrc_hbm.at[ebuf[b].at[1]], msg[b], semg[b]).wait()
                for g in range(CH // L):
                    sl = pl.ds(g * L, L)
                    lib[b][0, sl] = ebuf[b][0, sl]
                @plsc.parallel_loop(0, CH // L)
                def _scale(l, b=b):
                    iv = ebuf[b][2, pl.ds(l * L, L)]
                    vv = plsc.bitcast(iv, jnp.float32)
                    base = l * L
                    for i0 in range(0, L, 4):
                        rows = [
                            [msg[b][base + i0 + i, pl.ds(j * L, L)]
                             for j in range(4)]
                            for i in range(4)
                        ]
                        for i in range(4):
                            v = vv[i0 + i]
                            for j in range(4):
                                sl = pl.ds(j * L, L)
                                msg[b][base + i0 + i, sl] = rows[i][j] * v
                pltpu.async_copy(msg[b], acc.at[lib[b].at[0]], sems[b],
                                 add=True)

            @pl.when(ch + 3 < nn)
            def _():
                pltpu.async_copy(echunk(ch + 3), ebuf[b], semi[b])
            @pl.when(ch + 2 < nn)
            def _():
                b2 = (b + 2) % 3
                pltpu.make_async_copy(
                    part_hbm.at[0, 0, 0], ebuf[b2], semi[b2]).wait()

                @pl.when(ch >= 1)
                def _():
                    pltpu.make_async_copy(
                        msg[b2], acc.at[lib[b2].at[0]], sems[b2]).wait()
                pltpu.async_copy(
                    xsrc_hbm.at[ebuf[b2].at[1]], msg[b2], semg[b2])
        return 0

    lax.fori_loop(0, lax.div(nn + 2, 3), _tri, 0)
    for q in range(3):
        pltpu.make_async_copy(msg[q], acc.at[lib[q].at[0]], sems[q]).wait()
    plsc.subcore_barrier()

    if second_layer:
        def _wb(k, _):
            loc = s * ROWS_PER_TILE + k * WB
            gbase = base_row + loc
            pltpu.sync_copy(acc.at[pl.ds(loc, WB)], msg0.at[pl.ds(0, WB)])
            pltpu.sync_copy(xadd_hbm.at[pl.ds(gbase, WB)],
                            msg1.at[pl.ds(0, WB)])
            pltpu.sync_copy(yadd_hbm.at[pl.ds(gbase, WB)],
                            msg2.at[pl.ds(0, WB)])

            @plsc.parallel_loop(0, WB)
            def _add(r):
                for j in range(4):
                    sl = pl.ds(j * L, L)
                    msg0[r, sl] = msg0[r, sl] + msg1[r, sl] + msg2[r, sl]
            pltpu.sync_copy(msg0.at[pl.ds(0, WB)], out_hbm.at[pl.ds(gbase, WB)])
            return 0
        lax.fori_loop(0, ROWS_PER_TILE // WB, _wb, 0)
    else:
        loc = s * ROWS_PER_TILE
        pltpu.sync_copy(acc.at[pl.ds(loc, ROWS_PER_TILE)],
                        out_hbm.at[pl.ds(base_row + loc, ROWS_PER_TILE)])


def _make_spmm(second_layer):
    scratch = [
        pltpu.VMEM((3, CH), jnp.int32),
        pltpu.VMEM((3, CH), jnp.int32),
        pltpu.VMEM((3, CH), jnp.int32),
        pltpu.VMEM((CH, D), jnp.float32),
        pltpu.VMEM((CH, D), jnp.float32),
        pltpu.VMEM((CH, D), jnp.float32),
        pltpu.VMEM((1, CH), jnp.int32),
        pltpu.VMEM((1, CH), jnp.int32),
        pltpu.VMEM((1, CH), jnp.int32),
        pltpu.VMEM((2, L), jnp.int32),
    ]
    scratch += [pltpu.SemaphoreType.DMA] * 9
    scratch += [pltpu.VMEM_SHARED((ACC_ROWS, D), jnp.float32)]
    mesh = plsc.VectorSubcoreMesh(
        core_axis_name="c", subcore_axis_name="s",
        num_cores=NC, num_subcores=NS)
    return pl.kernel(
        functools.partial(_spmm_body, second_layer),
        out_type=jax.ShapeDtypeStruct((NPAD, D), jnp.float32),
        mesh=mesh,
        scratch_types=scratch,
        compiler_params=pltpu.CompilerParams(
            use_tc_tiling_on_sc=False, needs_layout_passes=False),
        name="spmm_layer2" if second_layer else "spmm_layer1",
    )


@jax.jit
def kernel(edge_index, adj_values, uEmbeds):
    ei = edge_index.astype(jnp.int32)
    x_pad = jnp.pad(uEmbeds, ((0, NPAD - N), (0, 0)))

    part, cnt = _make_partition()(ei, adj_values)
    y1 = _make_spmm(False)(part, cnt, x_pad)
    out = _make_spmm(True)(part, cnt, y1, x_pad, y1)
    return out[:N]
